# baseline (speedup 1.0000x reference)
import jax
import jax.numpy as jnp
from jax.experimental import pallas as pl

INPUT_DIM = 768
VQ_DIM = 256
NUM_EMBED = 1024
SPLIT_NUM = 4


def _mk(x_ref, proj_ref, cbn_ref, out_ref):
    feat = jnp.dot(x_ref[...], proj_ref[...])
    norm = jnp.sqrt(jnp.sum(feat * feat, axis=1, keepdims=True))
    feat = feat / norm
    sim = jax.lax.dot_general(feat, cbn_ref[...], (((1,), (1,)), ((), ())))
    lane = jax.lax.broadcasted_iota(jnp.int32, sim.shape, 1)
    m = jnp.max(sim, axis=1, keepdims=True)
    idx = jnp.min(jnp.where(sim == m, lane, NUM_EMBED), axis=1, keepdims=True)
    out_ref[...] = idx.astype(jnp.int32)


def kernel(x, projector, codebook, random_matrix):
    b, t, n = x.shape
    x_fft = jnp.fft.rfft(x, axis=-1)
    x_fft_subsample = [x_fft * (random_matrix == i) for i in range(SPLIT_NUM)]
    x_fft_subsample = jnp.stack(x_fft_subsample, axis=2)
    x_recon = jnp.fft.irfft(x_fft_subsample, n=INPUT_DIM, axis=-1)
    rows = b * t * SPLIT_NUM
    xr = x_recon.reshape(rows, n)
    cbn = (codebook / jnp.linalg.norm(codebook, axis=-1, keepdims=True)
           ).reshape(NUM_EMBED, VQ_DIM)

    blk = 1024
    out = pl.pallas_call(
        _mk,
        grid=(rows // blk,),
        in_specs=[
            pl.BlockSpec((blk, n), lambda r: (r, 0)),
            pl.BlockSpec((INPUT_DIM, VQ_DIM), lambda r: (0, 0)),
            pl.BlockSpec((NUM_EMBED, VQ_DIM), lambda r: (0, 0)),
        ],
        out_specs=pl.BlockSpec((blk, 1), lambda r: (r, 0)),
        out_shape=jax.ShapeDtypeStruct((rows, 1), jnp.int32),
    )(xr, projector, cbn)
    return out.reshape(b, t, SPLIT_NUM, 1)


# speedup vs baseline: 8.2603x; 8.2603x over previous
import jax
import jax.numpy as jnp
import numpy as np
from jax.experimental import pallas as pl

INPUT_DIM = 768
VQ_DIM = 256
NUM_EMBED = 1024
SPLIT_NUM = 4
FREQ = INPUT_DIM // 2 + 1
FPAD = 512

_k = np.arange(FPAD, dtype=np.int64)[:, None]
_i = np.arange(INPUT_DIM, dtype=np.int64)[None, :]
_ang = 2.0 * np.pi * ((_k * _i) % INPUT_DIM) / INPUT_DIM
_COS = np.cos(_ang).astype(np.float32)
_SIN = np.sin(_ang).astype(np.float32)
_HI = jax.lax.Precision.HIGHEST


def _fold_kernel(cos_ref, sin_ref, rm_ref, cb_ref, b_ref, cbn_ref):
    k = jax.lax.broadcasted_iota(jnp.int32, (FPAD, 1), 0)
    w = jnp.where((k == 0) | (k == INPUT_DIM // 2), 1.0, 2.0)
    rm = rm_ref[...]
    inv_n = 1.0 / INPUT_DIM
    c = cos_ref[...]
    s = sin_ref[...]
    for sp in range(SPLIT_NUM):
        ds = jnp.where(rm == sp, w, 0.0)
        b = jax.lax.dot_general(c, ds * c, (((0,), (0,)), ((), ())),
                                precision=_HI)
        b += jax.lax.dot_general(s, ds * s, (((0,), (0,)), ((), ())),
                                 precision=_HI)
        b_ref[:, sp * INPUT_DIM:(sp + 1) * INPUT_DIM] = b * inv_n

    cb = cb_ref[...]
    cbn_ref[...] = cb / jnp.sqrt(jnp.sum(cb * cb, axis=1, keepdims=True))


def _main_kernel(x_ref, b_ref, proj_ref, cbn_ref, out_ref):
    recon = jnp.dot(x_ref[...], b_ref[...], precision=_HI)
    proj = proj_ref[...]
    cbn = cbn_ref[...]
    lane = jax.lax.broadcasted_iota(
        jnp.int32, (x_ref.shape[0], NUM_EMBED), 1)
    cols = []
    for sp in range(SPLIT_NUM):
        feat = jnp.dot(recon[:, sp * INPUT_DIM:(sp + 1) * INPUT_DIM], proj)
        feat = feat / jnp.sqrt(jnp.sum(feat * feat, axis=1, keepdims=True))
        sim = jax.lax.dot_general(feat, cbn, (((1,), (1,)), ((), ())))
        m = jnp.max(sim, axis=1, keepdims=True)
        idx = jnp.min(jnp.where(sim == m, lane, NUM_EMBED),
                      axis=1, keepdims=True)
        cols.append(idx)
    out_ref[...] = jnp.concatenate(cols, axis=1).astype(jnp.int32)


def kernel(x, projector, codebook, random_matrix):
    b, t, n = x.shape
    rows = b * t
    xr = x.reshape(rows, n)
    cb = codebook.reshape(NUM_EMBED, VQ_DIM)
    rm = jnp.full((FPAD, 1), -1, dtype=jnp.int32)
    rm = rm.at[:FREQ, 0].set(random_matrix.astype(jnp.int32))

    bmat, cbn = pl.pallas_call(
        _fold_kernel,
        out_shape=(
            jax.ShapeDtypeStruct((INPUT_DIM, SPLIT_NUM * INPUT_DIM),
                                 jnp.float32),
            jax.ShapeDtypeStruct((NUM_EMBED, VQ_DIM), jnp.float32),
        ),
    )(jnp.asarray(_COS), jnp.asarray(_SIN), rm, cb)

    blk = 512
    out = pl.pallas_call(
        _main_kernel,
        grid=(rows // blk,),
        in_specs=[
            pl.BlockSpec((blk, n), lambda r: (r, 0)),
            pl.BlockSpec((INPUT_DIM, SPLIT_NUM * INPUT_DIM),
                         lambda r: (0, 0)),
            pl.BlockSpec((INPUT_DIM, VQ_DIM), lambda r: (0, 0)),
            pl.BlockSpec((NUM_EMBED, VQ_DIM), lambda r: (0, 0)),
        ],
        out_specs=pl.BlockSpec((blk, SPLIT_NUM), lambda r: (r, 0)),
        out_shape=jax.ShapeDtypeStruct((rows, SPLIT_NUM), jnp.int32),
    )(xr, bmat, projector, cbn)

    return out.reshape(b, t, SPLIT_NUM, 1)
